# baseline (speedup 1.0000x reference)
import functools

import jax
import jax.numpy as jnp
from jax import lax
from jax.experimental import pallas as pl
from jax.experimental.pallas import tpu as pltpu
from jax.experimental.pallas import tpu_sc as plsc

N_TABLE = 10000
D = 128
N_ELEMS = 320000
N_SEG = 10000

NC = 2
NS = 16
CHUNK = 40
ELEMS_PER_TILE = N_ELEMS // (NC * NS)
N_CHUNKS = ELEMS_PER_TILE // CHUNK
N_TRIPS = (N_CHUNKS - 4) // 3
SEG_PER_TILE = N_SEG // NS
ZROWS = 25
NZERO = SEG_PER_TILE // ZROWS


def _sc_partials(table, idx_flat, seg_flat):
    mesh = plsc.VectorSubcoreMesh(core_axis_name="c", subcore_axis_name="s")

    @functools.partial(
        pl.kernel,
        mesh=mesh,
        out_type=[
            jax.ShapeDtypeStruct((NC, NS, SEG_PER_TILE, D), jnp.float32),
            jax.ShapeDtypeStruct((NC, NS, SEG_PER_TILE, D), jnp.float32),
        ],
        scratch_types=[
            pltpu.VMEM_SHARED((N_SEG, D), jnp.float32),
            pltpu.VMEM((ELEMS_PER_TILE,), jnp.int32),
            pltpu.VMEM((ELEMS_PER_TILE,), jnp.int32),
            pltpu.VMEM((CHUNK, D), jnp.float32),
            pltpu.VMEM((CHUNK, D), jnp.float32),
            pltpu.VMEM((CHUNK, D), jnp.float32),
            pltpu.VMEM((ZROWS, D), jnp.float32),
            pltpu.SemaphoreType.DMA,
            pltpu.SemaphoreType.DMA,
            pltpu.SemaphoreType.DMA,
            pltpu.SemaphoreType.DMA,
            pltpu.SemaphoreType.DMA,
            pltpu.SemaphoreType.DMA,
            pltpu.SemaphoreType.DMA,
        ],
    )
    def k(table_hbm, idx_hbm, seg_hbm, psum_hbm, pboth_hbm,
          acc_sh, idx_st, seg_st, rows_a, rows_b, rows_c, zrow_v,
          gsem_a, gsem_b, gsem_c, ssem_a, ssem_b, ssem_c, zsem):
        cid = lax.axis_index("c")
        sid = lax.axis_index("s")
        wid = cid * NS + sid
        seg_base = sid * SEG_PER_TILE

        z16 = jnp.zeros((16,), jnp.float32)
        one16 = jnp.ones((16,), jnp.float32)

        def fill(ref, nrows, val):
            def body(r, carry):
                for cb in range(D // 16):
                    ref[r, pl.ds(cb * 16, 16)] = val
                return carry
            lax.fori_loop(0, nrows, body, 0)

        ebase = wid * ELEMS_PER_TILE
        pltpu.sync_copy(idx_hbm.at[pl.ds(ebase, ELEMS_PER_TILE)], idx_st)
        pltpu.sync_copy(seg_hbm.at[pl.ds(ebase, ELEMS_PER_TILE)], seg_st)

        fill(zrow_v, ZROWS, z16)
        for j in range(NZERO):
            pltpu.async_copy(
                zrow_v, acc_sh.at[pl.ds(seg_base + j * ZROWS, ZROWS)], zsem)
        for j in range(NZERO):
            pltpu.make_async_copy(
                zrow_v, acc_sh.at[pl.ds(seg_base + j * ZROWS, ZROWS)],
                zsem).wait()
        plsc.subcore_barrier()

        def gather(c, rows_v, sem):
            idx = idx_st.at[pl.ds(c * CHUNK, CHUNK)]
            return pltpu.async_copy(table_hbm.at[idx], rows_v, sem)

        def gather_wait(c, rows_v, sem):
            idx = idx_st.at[pl.ds(c * CHUNK, CHUNK)]
            pltpu.make_async_copy(table_hbm.at[idx], rows_v, sem).wait()

        def scat_start(c, rows_v, sem):
            seg = seg_st.at[pl.ds(c * CHUNK, CHUNK)]
            pltpu.async_copy(rows_v, acc_sh.at[seg], sem, add=True)

        def scat_wait(c, rows_v, sem):
            seg = seg_st.at[pl.ds(c * CHUNK, CHUNK)]
            pltpu.make_async_copy(rows_v, acc_sh.at[seg], sem).wait()

        R = (rows_a, rows_b, rows_c)
        G = (gsem_a, gsem_b, gsem_c)
        S = (ssem_a, ssem_b, ssem_c)

        gather(0, rows_a, gsem_a)
        gather(1, rows_b, gsem_b)
        gather_wait(0, rows_a, gsem_a)
        scat_start(0, rows_a, ssem_a)
        gather(2, rows_c, gsem_c)

        def stage(c, k, with_gather=True):
            kn = (k + 2) % 3
            gather_wait(c, R[k], G[k])
            scat_start(c, R[k], S[k])
            if with_gather:
                scat_wait(c - 1, R[kn], S[kn])
                gather(c + 2, R[kn], G[kn])

        def sum_trip(t, carry):
            c = 3 * t
            stage(c + 1, 1)
            stage(c + 2, 2)
            stage(c + 3, 0)
            return carry

        lax.fori_loop(0, N_TRIPS, sum_trip, 0)
        for c in range(3 * N_TRIPS + 1, N_CHUNKS):
            stage(c, c % 3, with_gather=(c + 2 <= N_CHUNKS - 1))
        for c in range(N_CHUNKS - 3, N_CHUNKS):
            scat_wait(c, R[c % 3], S[c % 3])

        plsc.subcore_barrier()
        pltpu.sync_copy(acc_sh.at[pl.ds(seg_base, SEG_PER_TILE)],
                        psum_hbm.at[cid, sid])
        plsc.subcore_barrier()

        fill(rows_a, CHUNK, one16)

        def ones_scat(c, sem):
            seg = seg_st.at[pl.ds(c * CHUNK, CHUNK)]
            pltpu.async_copy(rows_a, acc_sh.at[seg], sem, add=True)

        def ones_wait(c, sem):
            seg = seg_st.at[pl.ds(c * CHUNK, CHUNK)]
            pltpu.make_async_copy(rows_a, acc_sh.at[seg], sem).wait()

        for c in range(4):
            ones_scat(c, ssem_a)

        def cnt_body(c, carry):
            ones_wait(c, ssem_a)
            ones_scat(c + 4, ssem_a)
            return carry

        lax.fori_loop(0, N_CHUNKS - 4, cnt_body, 0)
        for c in range(N_CHUNKS - 4, N_CHUNKS):
            ones_wait(c, ssem_a)

        plsc.subcore_barrier()
        pltpu.sync_copy(acc_sh.at[pl.ds(seg_base, SEG_PER_TILE)],
                        pboth_hbm.at[cid, sid])

    return k(table, idx_flat, seg_flat)


def _combine(psum, pboth):
    BLK = 2000

    def body(s_ref, b_ref, o_ref):
        s = s_ref[0] + s_ref[1]
        cnt = (b_ref[0] - s_ref[0]) + (b_ref[1] - s_ref[1])
        o_ref[...] = s / jnp.maximum(cnt, 1.0)

    return pl.pallas_call(
        body,
        grid=(N_SEG // BLK,),
        in_specs=[
            pl.BlockSpec((NC, BLK, D), lambda i: (0, i, 0)),
            pl.BlockSpec((NC, BLK, D), lambda i: (0, i, 0)),
        ],
        out_specs=pl.BlockSpec((BLK, D), lambda i: (i, 0)),
        out_shape=jax.ShapeDtypeStruct((N_SEG, D), jnp.float32),
    )(psum, pboth)


def kernel(chunk_features, flat_indices, segment_ids):
    psum, pboth = _sc_partials(chunk_features, flat_indices, segment_ids)
    return _combine(psum.reshape(NC, N_SEG, D), pboth.reshape(NC, N_SEG, D))


# speedup vs baseline: 1.0563x; 1.0563x over previous
import functools

import jax
import jax.numpy as jnp
from jax import lax
from jax.experimental import pallas as pl
from jax.experimental.pallas import tpu as pltpu
from jax.experimental.pallas import tpu_sc as plsc

N_TABLE = 10000
D = 128
N_ELEMS = 320000
N_SEG = 10000

NC = 2
NS = 16
CHUNK = 80
ELEMS_PER_TILE = N_ELEMS // (NC * NS)
N_CHUNKS = ELEMS_PER_TILE // CHUNK
N_PAIRS = (N_CHUNKS - 1) // 2
SEG_PER_TILE = N_SEG // NS
ZROWS = 25
NZERO = SEG_PER_TILE // ZROWS


def _sc_partials(table, idx_flat, seg_flat):
    mesh = plsc.VectorSubcoreMesh(core_axis_name="c", subcore_axis_name="s")

    @functools.partial(
        pl.kernel,
        mesh=mesh,
        out_type=[
            jax.ShapeDtypeStruct((NC, NS, SEG_PER_TILE, D), jnp.float32),
            jax.ShapeDtypeStruct((NC, NS, SEG_PER_TILE, D), jnp.float32),
        ],
        scratch_types=[
            pltpu.VMEM_SHARED((N_SEG, D), jnp.float32),
            pltpu.VMEM((ELEMS_PER_TILE,), jnp.int32),
            pltpu.VMEM((ELEMS_PER_TILE,), jnp.int32),
            pltpu.VMEM((CHUNK, D), jnp.float32),
            pltpu.VMEM((CHUNK, D), jnp.float32),
            pltpu.VMEM((ZROWS, D), jnp.float32),
            pltpu.SemaphoreType.DMA,
            pltpu.SemaphoreType.DMA,
            pltpu.SemaphoreType.DMA,
            pltpu.SemaphoreType.DMA,
            pltpu.SemaphoreType.DMA,
        ],
    )
    def k(table_hbm, idx_hbm, seg_hbm, psum_hbm, pboth_hbm,
          acc_sh, idx_st, seg_st, rows_a, rows_b, zrow_v,
          gsem_a, gsem_b, ssem_a, ssem_b, zsem):
        cid = lax.axis_index("c")
        sid = lax.axis_index("s")
        wid = cid * NS + sid
        seg_base = sid * SEG_PER_TILE

        z16 = jnp.zeros((16,), jnp.float32)
        one16 = jnp.ones((16,), jnp.float32)

        def fill(ref, nrows, val):
            def body(r, carry):
                for cb in range(D // 16):
                    ref[r, pl.ds(cb * 16, 16)] = val
                return carry
            lax.fori_loop(0, nrows, body, 0)

        ebase = wid * ELEMS_PER_TILE
        pltpu.sync_copy(idx_hbm.at[pl.ds(ebase, ELEMS_PER_TILE)], idx_st)
        pltpu.sync_copy(seg_hbm.at[pl.ds(ebase, ELEMS_PER_TILE)], seg_st)

        fill(zrow_v, ZROWS, z16)
        for j in range(NZERO):
            pltpu.async_copy(
                zrow_v, acc_sh.at[pl.ds(seg_base + j * ZROWS, ZROWS)], zsem)
        for j in range(NZERO):
            pltpu.make_async_copy(
                zrow_v, acc_sh.at[pl.ds(seg_base + j * ZROWS, ZROWS)],
                zsem).wait()
        plsc.subcore_barrier()

        def gather(c, rows_v, sem):
            idx = idx_st.at[pl.ds(c * CHUNK, CHUNK)]
            return pltpu.async_copy(table_hbm.at[idx], rows_v, sem)

        def gather_wait(c, rows_v, sem):
            idx = idx_st.at[pl.ds(c * CHUNK, CHUNK)]
            pltpu.make_async_copy(table_hbm.at[idx], rows_v, sem).wait()

        def scat(c, rows_v):
            seg = seg_st.at[pl.ds(c * CHUNK, CHUNK)]
            pltpu.sync_copy(rows_v, acc_sh.at[seg], add=True)

        gather(0, rows_a, gsem_a)

        def sum_pair(p, carry):
            gather(2 * p + 1, rows_b, gsem_b)
            gather_wait(2 * p, rows_a, gsem_a)
            scat(2 * p, rows_a)
            gather(2 * p + 2, rows_a, gsem_a)
            gather_wait(2 * p + 1, rows_b, gsem_b)
            scat(2 * p + 1, rows_b)
            return carry

        lax.fori_loop(0, N_PAIRS, sum_pair, 0)
        gather_wait(N_CHUNKS - 1, rows_a, gsem_a)
        scat(N_CHUNKS - 1, rows_a)

        plsc.subcore_barrier()
        pltpu.sync_copy(acc_sh.at[pl.ds(seg_base, SEG_PER_TILE)],
                        psum_hbm.at[cid, sid])
        plsc.subcore_barrier()

        fill(rows_a, CHUNK, one16)

        def ones_scat(c, sem):
            seg = seg_st.at[pl.ds(c * CHUNK, CHUNK)]
            pltpu.async_copy(rows_a, acc_sh.at[seg], sem, add=True)

        def ones_wait(c, sem):
            seg = seg_st.at[pl.ds(c * CHUNK, CHUNK)]
            pltpu.make_async_copy(rows_a, acc_sh.at[seg], sem).wait()

        for c in range(4):
            ones_scat(c, ssem_a)

        def cnt_body(c, carry):
            ones_wait(c, ssem_a)
            ones_scat(c + 4, ssem_a)
            return carry

        lax.fori_loop(0, N_CHUNKS - 4, cnt_body, 0)
        for c in range(N_CHUNKS - 4, N_CHUNKS):
            ones_wait(c, ssem_a)

        plsc.subcore_barrier()
        pltpu.sync_copy(acc_sh.at[pl.ds(seg_base, SEG_PER_TILE)],
                        pboth_hbm.at[cid, sid])

    return k(table, idx_flat, seg_flat)


def _combine(psum, pboth):
    BLK = 2000

    def body(s_ref, b_ref, o_ref):
        s = s_ref[0] + s_ref[1]
        cnt = (b_ref[0] - s_ref[0]) + (b_ref[1] - s_ref[1])
        o_ref[...] = s / jnp.maximum(cnt, 1.0)

    return pl.pallas_call(
        body,
        grid=(N_SEG // BLK,),
        in_specs=[
            pl.BlockSpec((NC, BLK, D), lambda i: (0, i, 0)),
            pl.BlockSpec((NC, BLK, D), lambda i: (0, i, 0)),
        ],
        out_specs=pl.BlockSpec((BLK, D), lambda i: (i, 0)),
        out_shape=jax.ShapeDtypeStruct((N_SEG, D), jnp.float32),
    )(psum, pboth)


def kernel(chunk_features, flat_indices, segment_ids):
    psum, pboth = _sc_partials(chunk_features, flat_indices, segment_ids)
    return _combine(psum.reshape(NC, N_SEG, D), pboth.reshape(NC, N_SEG, D))


# speedup vs baseline: 1.0774x; 1.0200x over previous
import functools

import jax
import jax.numpy as jnp
from jax import lax
from jax.experimental import pallas as pl
from jax.experimental.pallas import tpu as pltpu
from jax.experimental.pallas import tpu_sc as plsc

N_TABLE = 10000
D = 128
N_ELEMS = 320000
N_SEG = 10000

NC = 2
NS = 16
CHUNK = 80
ELEMS_PER_TILE = N_ELEMS // (NC * NS)
N_CHUNKS = ELEMS_PER_TILE // CHUNK
N_PAIRS = (N_CHUNKS - 1) // 2
SEG_PER_TILE = N_SEG // NS
ZROWS = 25
NZERO = SEG_PER_TILE // ZROWS


def _sc_partials(table, idx_flat, seg_flat):
    mesh = plsc.VectorSubcoreMesh(core_axis_name="c", subcore_axis_name="s")

    @functools.partial(
        pl.kernel,
        mesh=mesh,
        out_type=[
            jax.ShapeDtypeStruct((NC, NS, SEG_PER_TILE, D), jnp.float32),
            jax.ShapeDtypeStruct((NC, NS, SEG_PER_TILE, D), jnp.float32),
        ],
        scratch_types=[
            pltpu.VMEM_SHARED((N_SEG, D), jnp.float32),
            pltpu.VMEM((ELEMS_PER_TILE,), jnp.int32),
            pltpu.VMEM((ELEMS_PER_TILE,), jnp.int32),
            pltpu.VMEM((CHUNK, D), jnp.float32),
            pltpu.VMEM((CHUNK, D), jnp.float32),
            pltpu.VMEM((ZROWS, D), jnp.float32),
            pltpu.SemaphoreType.DMA,
            pltpu.SemaphoreType.DMA,
            pltpu.SemaphoreType.DMA,
            pltpu.SemaphoreType.DMA,
            pltpu.SemaphoreType.DMA,
        ],
    )
    def k(table_hbm, idx_hbm, seg_hbm, psum_hbm, pboth_hbm,
          acc_sh, idx_st, seg_st, rows_a, rows_b, zrow_v,
          gsem_a, gsem_b, ssem_a, ssem_b, zsem):
        cid = lax.axis_index("c")
        sid = lax.axis_index("s")
        wid = cid * NS + sid
        seg_base = sid * SEG_PER_TILE

        z16 = jnp.zeros((16,), jnp.float32)
        one16 = jnp.ones((16,), jnp.float32)

        def fill(ref, nrows, val):
            def body(r, carry):
                for cb in range(D // 16):
                    ref[r, pl.ds(cb * 16, 16)] = val
                return carry
            lax.fori_loop(0, nrows, body, 0)

        ebase = wid * ELEMS_PER_TILE
        pltpu.sync_copy(idx_hbm.at[pl.ds(ebase, ELEMS_PER_TILE)], idx_st)
        pltpu.sync_copy(seg_hbm.at[pl.ds(ebase, ELEMS_PER_TILE)], seg_st)

        fill(zrow_v, ZROWS, z16)
        for j in range(NZERO):
            pltpu.async_copy(
                zrow_v, acc_sh.at[pl.ds(seg_base + j * ZROWS, ZROWS)], zsem)
        for j in range(NZERO):
            pltpu.make_async_copy(
                zrow_v, acc_sh.at[pl.ds(seg_base + j * ZROWS, ZROWS)],
                zsem).wait()
        plsc.subcore_barrier()

        def gather(c, rows_v, sem):
            idx = idx_st.at[pl.ds(c * CHUNK, CHUNK)]
            return pltpu.async_copy(table_hbm.at[idx], rows_v, sem)

        def gather_wait(c, rows_v, sem):
            idx = idx_st.at[pl.ds(c * CHUNK, CHUNK)]
            pltpu.make_async_copy(table_hbm.at[idx], rows_v, sem).wait()

        def scat(c, rows_v):
            seg = seg_st.at[pl.ds(c * CHUNK, CHUNK)]
            pltpu.sync_copy(rows_v, acc_sh.at[seg], add=True)

        gather(0, rows_a, gsem_a)

        def sum_pair(p, carry):
            gather(2 * p + 1, rows_b, gsem_b)
            gather_wait(2 * p, rows_a, gsem_a)
            scat(2 * p, rows_a)
            gather(2 * p + 2, rows_a, gsem_a)
            gather_wait(2 * p + 1, rows_b, gsem_b)
            scat(2 * p + 1, rows_b)
            return carry

        lax.fori_loop(0, N_PAIRS, sum_pair, 0)
        gather_wait(N_CHUNKS - 1, rows_a, gsem_a)
        scat(N_CHUNKS - 1, rows_a)

        plsc.subcore_barrier()
        pltpu.sync_copy(acc_sh.at[pl.ds(seg_base, SEG_PER_TILE)],
                        psum_hbm.at[cid, sid])
        plsc.subcore_barrier()

        fill(rows_a, CHUNK, one16)

        def ones_scat(c, sem):
            seg = seg_st.at[pl.ds(c * CHUNK, CHUNK)]
            pltpu.async_copy(rows_a, acc_sh.at[seg], sem, add=True)

        def ones_wait(c, sem):
            seg = seg_st.at[pl.ds(c * CHUNK, CHUNK)]
            pltpu.make_async_copy(rows_a, acc_sh.at[seg], sem).wait()

        ones_scat(0, ssem_a)

        def cnt_pair(p, carry):
            ones_scat(2 * p + 1, ssem_b)
            ones_wait(2 * p, ssem_a)
            ones_scat(2 * p + 2, ssem_a)
            ones_wait(2 * p + 1, ssem_b)
            return carry

        lax.fori_loop(0, N_PAIRS, cnt_pair, 0)
        ones_wait(N_CHUNKS - 1, ssem_a)

        plsc.subcore_barrier()
        pltpu.sync_copy(acc_sh.at[pl.ds(seg_base, SEG_PER_TILE)],
                        pboth_hbm.at[cid, sid])

    return k(table, idx_flat, seg_flat)


def _combine(psum, pboth):
    BLK = 2000

    def body(s_ref, b_ref, o_ref):
        s = s_ref[0] + s_ref[1]
        cnt = (b_ref[0] - s_ref[0]) + (b_ref[1] - s_ref[1])
        o_ref[...] = s / jnp.maximum(cnt, 1.0)

    return pl.pallas_call(
        body,
        grid=(N_SEG // BLK,),
        in_specs=[
            pl.BlockSpec((NC, BLK, D), lambda i: (0, i, 0)),
            pl.BlockSpec((NC, BLK, D), lambda i: (0, i, 0)),
        ],
        out_specs=pl.BlockSpec((BLK, D), lambda i: (i, 0)),
        out_shape=jax.ShapeDtypeStruct((N_SEG, D), jnp.float32),
    )(psum, pboth)


def kernel(chunk_features, flat_indices, segment_ids):
    psum, pboth = _sc_partials(chunk_features, flat_indices, segment_ids)
    return _combine(psum.reshape(NC, N_SEG, D), pboth.reshape(NC, N_SEG, D))
